# baseline (speedup 1.0000x reference)
import jax
import jax.numpy as jnp
from jax.experimental import pallas as pl
from jax.experimental.pallas import tpu as pltpu

_EMB = 3072
_SMALL = 61
_SMALL_PAD = 128
_N = 1024
_M_BLK = 512


def _proj_kernel(x_ref, s_ref, we_ref, ws_ref, b_ref, m_ref, o_ref,
                 we16_ref, ws16_ref):
    @pl.when(pl.program_id(0) == 0)
    def _prep():
        we16_ref[...] = we_ref[...].astype(jnp.bfloat16)
        col = jax.lax.broadcasted_iota(jnp.int32, (_N, _SMALL_PAD), 1)
        ws16_ref[...] = jnp.where(col < _SMALL, ws_ref[...], 0.0).astype(jnp.bfloat16)

    x = x_ref[...].astype(jnp.bfloat16)
    acc = jax.lax.dot_general(
        x, we16_ref[...], (((1,), (1,)), ((), ())),
        preferred_element_type=jnp.float32)
    acc += jax.lax.dot_general(
        s_ref[...].astype(jnp.bfloat16), ws16_ref[...], (((1,), (1,)), ((), ())),
        preferred_element_type=jnp.float32)
    o_ref[...] = (acc + b_ref[...]) * m_ref[...]


def kernel(embeddings, visibility_scores, bbox_ltwh, keypoints_xyc, feats_masks, W, b):
    bsz, slen = feats_masks.shape
    m_rows = bsz * slen

    x = embeddings.reshape(m_rows, _EMB)
    small = jnp.concatenate(
        [visibility_scores.reshape(m_rows, 6),
         bbox_ltwh.reshape(m_rows, 4),
         keypoints_xyc.reshape(m_rows, 51),
         jnp.zeros((m_rows, _SMALL_PAD - _SMALL), jnp.float32)],
        axis=-1)
    mask = feats_masks.reshape(m_rows, 1).astype(jnp.float32)
    bias = b.reshape(1, _N)

    grid = (m_rows // _M_BLK,)
    out = pl.pallas_call(
        _proj_kernel,
        grid=grid,
        in_specs=[
            pl.BlockSpec((_M_BLK, _EMB), lambda m: (m, 0)),
            pl.BlockSpec((_M_BLK, _SMALL_PAD), lambda m: (m, 0)),
            pl.BlockSpec((_N, _EMB), lambda m: (0, 0)),
            pl.BlockSpec((_N, _SMALL_PAD), lambda m: (0, _EMB // _SMALL_PAD)),
            pl.BlockSpec((1, _N), lambda m: (0, 0)),
            pl.BlockSpec((_M_BLK, 1), lambda m: (m, 0)),
        ],
        out_specs=pl.BlockSpec((_M_BLK, _N), lambda m: (m, 0)),
        out_shape=jax.ShapeDtypeStruct((m_rows, _N), jnp.float32),
        scratch_shapes=[
            pltpu.VMEM((_N, _EMB), jnp.bfloat16),
            pltpu.VMEM((_N, _SMALL_PAD), jnp.bfloat16),
        ],
    )(x, small, W, W, bias, mask)

    return out.reshape(bsz, slen, _N)
